# baseline (speedup 1.0000x reference)
import functools

import jax
import jax.numpy as jnp
from jax import lax
from jax.experimental import pallas as pl
from jax.experimental.pallas import tpu as pltpu
from jax.experimental.pallas import tpu_sc as plsc

_H = 1024
_W = 1024
_STRIDE = 126
_GRID = 9
_C = 16
_NR = 21
_CHUNKS = _STRIDE // _NR
_ITEMS_PER_W = _C * 8 * _CHUNKS // 32
_STEPS = _ITEMS_PER_W // 2
_BUFP = 144


def _sc_body(x, out, bufs, rows, in_sem, out_sem):
    cid = lax.axis_index("c")
    sid = lax.axis_index("s")
    wid = cid * 16 + sid

    def _coords(item):
        a = wid * _ITEMS_PER_W + item
        c = a // (8 * _CHUNKS)
        rem = a % (8 * _CHUNKS)
        i = rem // _CHUNKS
        r0 = (rem % _CHUNKS) * _NR
        return c, i, r0

    def _in_copy(item, b):
        c, i, r0 = _coords(item)
        return pltpu.make_async_copy(
            x.at[pl.ds(i * _GRID, _GRID), c, pl.ds(r0, _NR), :],
            bufs.at[b, :, :, pl.ds(0, 128)],
            in_sem.at[b],
        )

    def _row_out_copy(item, b, r):
        c, i, r0 = _coords(item)
        h = i * _STRIDE + r0 + r
        return pltpu.make_async_copy(
            rows.at[b, r],
            out.at[c, h // 8, :, h % 8, :],
            out_sem.at[b],
        )

    def _assemble_rows(b, nrows, item=None):
        iota = lax.iota(jnp.int32, 16)

        @plsc.parallel_loop(0, nrows, 1, unroll=2)
        def _row(r):
            for v in range(64):
                w0 = 16 * v
                j = w0 // _STRIDE
                t, off = divmod(w0, 128)
                a = bufs[b, j, r, pl.ds(w0 - j * _STRIDE, 16)]
                bound = (j + 1) * _STRIDE
                if j < 8 and w0 + 16 > bound:
                    d = bound - w0
                    nxt = bufs[b, j + 1, r, pl.ds(0, 16)]
                    idx = jnp.maximum(iota - d, 0)
                    shifted = lax.gather(
                        nxt,
                        idx[:, None],
                        lax.GatherDimensionNumbers(
                            offset_dims=(),
                            collapsed_slice_dims=(0,),
                            start_index_map=(0,),
                        ),
                        (1,),
                        mode=lax.GatherScatterMode.PROMISE_IN_BOUNDS,
                    )
                    a = jnp.where(iota < d, a, shifted)
                rows[b, r, t, pl.ds(off, 16)] = a
            if item is not None:
                _row_out_copy(item, b, r).start()

    def _drain_out(b, nrows):
        pltpu.make_async_copy(
            x.at[pl.ds(0, nrows), 0, pl.ds(0, 8), :],
            rows.at[b, pl.ds(0, nrows)],
            out_sem.at[b],
        ).wait()

    _in_copy(0, 0).start()
    _in_copy(1, 1).start()

    def _pipe(t, carry):
        for b in range(2):
            item = 2 * t + b

            @pl.when(item >= 2)
            def _wait_out():
                _drain_out(b, _NR)

            _in_copy(item, b).wait()
            _assemble_rows(b, _NR, item)

            @pl.when(item + 2 < _ITEMS_PER_W)
            def _next_in():
                _in_copy(item + 2, b).start()

        return carry

    lax.fori_loop(0, _STEPS, _pipe, 0)
    _drain_out(0, _NR)
    _drain_out(1, _NR)

    @pl.when(wid < _C)
    def _():
        c = wid
        pltpu.sync_copy(
            x.at[pl.ds(8 * _GRID, _GRID), c, pl.ds(0, 16), :],
            bufs.at[0, :, pl.ds(0, 16), pl.ds(0, 128)],
        )
        _assemble_rows(0, 16)
        for r in range(16):
            h = 8 * _STRIDE + r
            pltpu.make_async_copy(
                rows.at[0, r],
                out.at[c, h // 8, :, h % 8, :],
                out_sem.at[0],
            ).start()
        _drain_out(0, 16)


_sc_kernel = functools.partial(
    pl.kernel,
    out_type=jax.ShapeDtypeStruct((_C, _H // 8, 8, 8, 128), jnp.float32),
    mesh=plsc.VectorSubcoreMesh(core_axis_name="c", subcore_axis_name="s"),
    scratch_types=[
        pltpu.VMEM((2, _GRID, _NR, _BUFP), jnp.float32),
        pltpu.VMEM((2, _NR, 8, 128), jnp.float32),
        pltpu.SemaphoreType.DMA((2,)),
        pltpu.SemaphoreType.DMA((2,)),
    ],
    compiler_params=pltpu.CompilerParams(use_tc_tiling_on_sc=False),
)(_sc_body)


def kernel(x):
    o = _sc_kernel(x)
    o = o.transpose(0, 1, 3, 2, 4)
    return o.reshape(_C, _H, _W)


# speedup vs baseline: 1.2717x; 1.2717x over previous
import functools

import jax
import jax.numpy as jnp
from jax import lax
from jax.experimental import pallas as pl
from jax.experimental.pallas import tpu as pltpu
from jax.experimental.pallas import tpu_sc as plsc

_H = 1024
_W = 1024
_STRIDE = 126
_GRID = 9
_C = 16
_NR = 21
_CHUNKS = _STRIDE // _NR
_ITEMS_PER_W = _C * 8 * _CHUNKS // 32
_STEPS = _ITEMS_PER_W // 2
_BUFP = 144


def _sc_body(x, out, bufs, rows, in_sem, out_sem):
    cid = lax.axis_index("c")
    sid = lax.axis_index("s")
    wid = cid * 16 + sid

    def _coords(item):
        a = wid * _ITEMS_PER_W + item
        c = a // (8 * _CHUNKS)
        rem = a % (8 * _CHUNKS)
        i = rem // _CHUNKS
        r0 = (rem % _CHUNKS) * _NR
        return c, i, r0

    def _in_copy(item, b):
        c, i, r0 = _coords(item)
        return pltpu.make_async_copy(
            x.at[pl.ds(i * _GRID, _GRID), c, pl.ds(r0, _NR), :],
            bufs.at[b, :, :, pl.ds(0, 128)],
            in_sem.at[b],
        )

    def _row_out_copy(item, b, r):
        c, i, r0 = _coords(item)
        h = i * _STRIDE + r0 + r
        return pltpu.make_async_copy(
            rows.at[b, r],
            out.at[c, h // 8, :, h % 8, :],
            out_sem.at[b],
        )

    def _assemble_rows(b, nrows, item=None):
        iota = lax.iota(jnp.int32, 16)

        @plsc.parallel_loop(0, nrows, 1)
        def _row(r):
            for v in range(64):
                w0 = 16 * v
                j = w0 // _STRIDE
                t, off = divmod(w0, 128)
                a = bufs[b, j, r, pl.ds(w0 - j * _STRIDE, 16)]
                bound = (j + 1) * _STRIDE
                if j < 8 and w0 + 16 > bound:
                    d = bound - w0
                    nxt = bufs[b, j + 1, r, pl.ds(0, 16)]
                    idx = jnp.maximum(iota - d, 0)
                    shifted = lax.gather(
                        nxt,
                        idx[:, None],
                        lax.GatherDimensionNumbers(
                            offset_dims=(),
                            collapsed_slice_dims=(0,),
                            start_index_map=(0,),
                        ),
                        (1,),
                        mode=lax.GatherScatterMode.PROMISE_IN_BOUNDS,
                    )
                    a = jnp.where(iota < d, a, shifted)
                rows[b, r, t, pl.ds(off, 16)] = a
            if item is not None:
                _row_out_copy(item, b, r).start()

    def _drain_out(b, nrows):
        pltpu.make_async_copy(
            x.at[pl.ds(0, nrows), 0, pl.ds(0, 8), :],
            rows.at[b, pl.ds(0, nrows)],
            out_sem.at[b],
        ).wait()

    _in_copy(0, 0).start()
    _in_copy(1, 1).start()

    def _pipe(t, carry):
        for b in range(2):
            item = 2 * t + b

            @pl.when(item >= 2)
            def _wait_out():
                _drain_out(b, _NR)

            _in_copy(item, b).wait()
            _assemble_rows(b, _NR, item)

            @pl.when(item + 2 < _ITEMS_PER_W)
            def _next_in():
                _in_copy(item + 2, b).start()

        return carry

    lax.fori_loop(0, _STEPS, _pipe, 0)
    _drain_out(0, _NR)
    _drain_out(1, _NR)

    @pl.when(wid < _C)
    def _():
        c = wid
        pltpu.sync_copy(
            x.at[pl.ds(8 * _GRID, _GRID), c, pl.ds(0, 16), :],
            bufs.at[0, :, pl.ds(0, 16), pl.ds(0, 128)],
        )
        _assemble_rows(0, 16)
        for r in range(16):
            h = 8 * _STRIDE + r
            pltpu.make_async_copy(
                rows.at[0, r],
                out.at[c, h // 8, :, h % 8, :],
                out_sem.at[0],
            ).start()
        _drain_out(0, 16)


_sc_kernel = functools.partial(
    pl.kernel,
    out_type=jax.ShapeDtypeStruct((_C, _H // 8, 8, 8, 128), jnp.float32),
    mesh=plsc.VectorSubcoreMesh(core_axis_name="c", subcore_axis_name="s"),
    scratch_types=[
        pltpu.VMEM((2, _GRID, _NR, _BUFP), jnp.float32),
        pltpu.VMEM((2, _NR, 8, 128), jnp.float32),
        pltpu.SemaphoreType.DMA((2,)),
        pltpu.SemaphoreType.DMA((2,)),
    ],
    compiler_params=pltpu.CompilerParams(use_tc_tiling_on_sc=False),
)(_sc_body)


def kernel(x):
    o = _sc_kernel(x)
    o = o.transpose(0, 1, 3, 2, 4)
    return o.reshape(_C, _H, _W)


# speedup vs baseline: 1.2907x; 1.0149x over previous
import functools

import jax
import jax.numpy as jnp
from jax import lax
from jax.experimental import pallas as pl
from jax.experimental.pallas import tpu as pltpu
from jax.experimental.pallas import tpu_sc as plsc

_H = 1024
_W = 1024
_STRIDE = 126
_GRID = 9
_C = 16
_NR = 21
_CHUNKS = _STRIDE // _NR
_ITEMS_PER_W = _C * 8 * _CHUNKS // 32
_STEPS = _ITEMS_PER_W // 2
_BUFP = 144


def _sc_body(x, out, bufs, rows, in_sem, out_sem):
    cid = lax.axis_index("c")
    sid = lax.axis_index("s")
    wid = cid * 16 + sid

    def _coords(item):
        a = wid * _ITEMS_PER_W + item
        c = a // (8 * _CHUNKS)
        rem = a % (8 * _CHUNKS)
        i = rem // _CHUNKS
        r0 = (rem % _CHUNKS) * _NR
        return c, i, r0

    def _in_copy(item, b):
        c, i, r0 = _coords(item)
        return pltpu.make_async_copy(
            x.at[pl.ds(i * _GRID, _GRID), c, pl.ds(r0, _NR), :],
            bufs.at[b, :, :, pl.ds(0, 128)],
            in_sem.at[b],
        )

    def _row_out_copy(item, b, r):
        c, i, r0 = _coords(item)
        h = i * _STRIDE + r0 + r
        return pltpu.make_async_copy(
            rows.at[b, r],
            out.at[c, h // 8, :, h % 8, :],
            out_sem.at[b],
        )

    def _assemble_rows(b, nrows, item=None):
        iota = lax.iota(jnp.int32, 16)

        @plsc.parallel_loop(0, nrows, 1)
        def _row(r):
            for v in range(64):
                w0 = 16 * v
                j = w0 // _STRIDE
                t, off = divmod(w0, 128)
                a = bufs[b, j, r, pl.ds(w0 - j * _STRIDE, 16)]
                bound = (j + 1) * _STRIDE
                if j < 8 and w0 + 16 > bound:
                    d = bound - w0
                    nxt = bufs[b, j + 1, r, pl.ds(0, 16)]
                    idx = jnp.maximum(iota - d, 0)
                    shifted = lax.gather(
                        nxt,
                        idx[:, None],
                        lax.GatherDimensionNumbers(
                            offset_dims=(),
                            collapsed_slice_dims=(0,),
                            start_index_map=(0,),
                        ),
                        (1,),
                        mode=lax.GatherScatterMode.PROMISE_IN_BOUNDS,
                    )
                    a = jnp.where(iota < d, a, shifted)
                rows[b, r, t, pl.ds(off, 16)] = a
            if item is not None:
                _row_out_copy(item, b, r).start()

    def _drain_out(b, nrows):
        pltpu.make_async_copy(
            x.at[pl.ds(0, nrows), 0, pl.ds(0, 8), :],
            rows.at[b, pl.ds(0, nrows)],
            out_sem.at[b],
        ).wait()

    _in_copy(0, 0).start()
    _in_copy(1, 1).start()

    def _pipe(t, carry):
        for b in range(2):
            item = 2 * t + b

            @pl.when(item >= 2)
            def _wait_out():
                _drain_out(b, _NR)

            _in_copy(item, b).wait()
            _assemble_rows(b, _NR, item)

            @pl.when(item + 2 < _ITEMS_PER_W)
            def _next_in():
                _in_copy(item + 2, b).start()

        return carry

    lax.fori_loop(0, _STEPS, _pipe, 0)
    _drain_out(0, _NR)
    _drain_out(1, _NR)

    c8 = wid // 2
    rb = 8 * (wid % 2)
    pltpu.sync_copy(
        x.at[pl.ds(8 * _GRID, _GRID), c8, pl.ds(rb, 8), :],
        bufs.at[0, :, pl.ds(0, 8), pl.ds(0, 128)],
    )
    _assemble_rows(0, 8)
    for r in range(8):
        h = 8 * _STRIDE + rb + r
        pltpu.make_async_copy(
            rows.at[0, r],
            out.at[c8, h // 8, :, h % 8, :],
            out_sem.at[0],
        ).start()
    _drain_out(0, 8)


_sc_kernel = functools.partial(
    pl.kernel,
    out_type=jax.ShapeDtypeStruct((_C, _H // 8, 8, 8, 128), jnp.float32),
    mesh=plsc.VectorSubcoreMesh(core_axis_name="c", subcore_axis_name="s"),
    scratch_types=[
        pltpu.VMEM((2, _GRID, _NR, _BUFP), jnp.float32),
        pltpu.VMEM((2, _NR, 8, 128), jnp.float32),
        pltpu.SemaphoreType.DMA((2,)),
        pltpu.SemaphoreType.DMA((2,)),
    ],
    compiler_params=pltpu.CompilerParams(use_tc_tiling_on_sc=False),
)(_sc_body)


def kernel(x):
    o = _sc_kernel(x)
    o = o.transpose(0, 1, 3, 2, 4)
    return o.reshape(_C, _H, _W)
